# baseline (speedup 1.0000x reference)
import functools

import jax
import jax.numpy as jnp
from jax import lax
from jax.experimental import pallas as pl
from jax.experimental.pallas import tpu as pltpu
from jax.experimental.pallas import tpu_sc as plsc

_VOCAB = 100000
_DIM = 64
_PDIM = 128
_BATCH = 4096
_CTX = 20

_PREP_BLK = 20480


def _prep_body(tt_ref, out_ref):
    i = pl.program_id(0)
    t = tt_ref[...]
    norm = jnp.sqrt(jnp.sum(t * t, axis=0, keepdims=True))
    scale = jnp.where(norm > 1.0, 1.0 / (norm + 1e-7), 1.0)
    col = i * _PREP_BLK + lax.broadcasted_iota(jnp.int32, t.shape, 1)
    scaled = jnp.where(col == 0, 0.0, t * scale)
    out_ref[:, : _DIM] = scaled.T


def _prep(tableT):
    return pl.pallas_call(
        _prep_body,
        grid=(-(-_VOCAB // _PREP_BLK),),
        in_specs=[pl.BlockSpec((_DIM, _PREP_BLK), lambda i: (0, i))],
        out_specs=pl.BlockSpec((_PREP_BLK, _PDIM), lambda i: (i, 0)),
        out_shape=jax.ShapeDtypeStruct((_VOCAB, _PDIM), jnp.float32),
    )(tableT)


_NW = 32
_BPW = _BATCH // _NW
_CHUNK = 16
_NCHUNK = _BPW // _CHUNK
_GROWS = _CHUNK * _CTX
_IDXALL = _BPW * _CTX // 128


def _gather_mean_body(tbl_hbm, idx_hbm, out_hbm, idx_v, rows_v, out_v,
                      sem0, sem1):
    wid = lax.axis_index("s") * 2 + lax.axis_index("c")
    pltpu.sync_copy(idx_hbm.at[wid], idx_v)
    sems = (sem0, sem1)

    def fire(c):
        slot = c % 2
        cps = []
        pos = c * _GROWS
        dst = slot * _GROWS
        while pos < (c + 1) * _GROWS:
            row, off = pos // 128, pos % 128
            n = min(128 - off, (c + 1) * _GROWS - pos)
            src = idx_v.at[row] if (off == 0 and n == 128) \
                else idx_v.at[row].at[pl.ds(off, n)]
            cps.append(
                pltpu.async_copy(
                    tbl_hbm.at[src],
                    rows_v.at[pl.ds(dst, n)],
                    sems[slot],
                )
            )
            pos += n
            dst += n
        return cps

    pending = fire(0)
    for c in range(_NCHUNK):
        nxt = fire(c + 1) if c + 1 < _NCHUNK else []
        for cp in pending:
            cp.wait()
        slot_base = (c % 2) * _GROWS

        def body(r, _):
            for d in range(_DIM // 16):
                acc = rows_v[slot_base + r * _CTX, pl.ds(d * 16, 16)]
                for j in range(1, _CTX):
                    acc = acc + rows_v[slot_base + r * _CTX + j,
                                       pl.ds(d * 16, 16)]
                out_v[c * _CHUNK + r, pl.ds(d * 16, 16)] = acc * (1.0 / _CTX)
            return 0

        lax.fori_loop(0, _CHUNK, body, 0)
        pending = nxt
    pltpu.sync_copy(out_v, out_hbm.at[pl.ds(wid * _BPW, _BPW)])


def _gather_mean(tbl3, x3):
    fn = functools.partial(
        pl.kernel,
        mesh=plsc.VectorSubcoreMesh(core_axis_name="c", subcore_axis_name="s"),
        out_type=jax.ShapeDtypeStruct((_BATCH, _DIM), jnp.float32),
        compiler_params=pltpu.CompilerParams(use_tc_tiling_on_sc=False),
        scratch_types=[
            pltpu.VMEM((_IDXALL, 128), jnp.int32),
            pltpu.VMEM((2 * _GROWS, _PDIM), jnp.float32),
            pltpu.VMEM((_BPW, _DIM), jnp.float32),
            pltpu.SemaphoreType.DMA,
            pltpu.SemaphoreType.DMA,
        ],
    )(_gather_mean_body)
    return fn(tbl3, x3)


_VT = 1024
_NVT = -(-_VOCAB // _VT)


def _linear_body(h_ref, wt_ref, b_ref, out_ref):
    acc = lax.dot_general(
        wt_ref[...], h_ref[...],
        (((0,), (1,)), ((), ())),
        preferred_element_type=jnp.float32,
    )
    bcol = jnp.transpose(b_ref[...])
    out_ref[...] = acc + bcol


def _linear(h, WT, brow):
    return pl.pallas_call(
        _linear_body,
        grid=(_NVT,),
        in_specs=[
            pl.BlockSpec((_BATCH, _DIM), lambda i: (0, 0)),
            pl.BlockSpec((_DIM, _VT), lambda i: (0, i)),
            pl.BlockSpec((1, _VT), lambda i: (0, i)),
        ],
        out_specs=pl.BlockSpec((_VT, _BATCH), lambda i: (i, 0)),
        out_shape=jax.ShapeDtypeStruct((_VOCAB, _BATCH), jnp.float32),
    )(h, WT, brow)


def kernel(x, table, W, b):
    x3 = x.astype(jnp.int32).reshape(_NW, _IDXALL, 128)
    tbl3 = _prep(table.T)
    h = _gather_mean(tbl3, x3)
    outT = _linear(h, W.T, b.reshape(1, _VOCAB))
    return outT.T


# speedup vs baseline: 1.0098x; 1.0098x over previous
import functools

import jax
import jax.numpy as jnp
from jax import lax
from jax.experimental import pallas as pl
from jax.experimental.pallas import tpu as pltpu
from jax.experimental.pallas import tpu_sc as plsc

_VOCAB = 100000
_DIM = 64
_PDIM = 128
_BATCH = 4096
_CTX = 20

_PREP_BLK = 12800


def _prep_body(tt_ref, out_ref):
    i = pl.program_id(0)
    t = tt_ref[...]
    norm = jnp.sqrt(jnp.sum(t * t, axis=0, keepdims=True))
    scale = jnp.where(norm > 1.0, 1.0 / (norm + 1e-7), 1.0)
    col = i * _PREP_BLK + lax.broadcasted_iota(jnp.int32, t.shape, 1)
    scaled = jnp.where(col == 0, 0.0, t * scale)
    out_ref[:, : _DIM] = scaled.T


def _prep(tableT):
    return pl.pallas_call(
        _prep_body,
        grid=(-(-_VOCAB // _PREP_BLK),),
        in_specs=[pl.BlockSpec((_DIM, _PREP_BLK), lambda i: (0, i))],
        out_specs=pl.BlockSpec((_PREP_BLK, _PDIM), lambda i: (i, 0)),
        out_shape=jax.ShapeDtypeStruct((_VOCAB, _PDIM), jnp.float32),
    )(tableT)


_NW = 32
_BPW = _BATCH // _NW
_CHUNK = 16
_NCHUNK = _BPW // _CHUNK
_GROWS = _CHUNK * _CTX
_IDXALL = _BPW * _CTX // 128


def _gather_mean_body(tbl_hbm, idx_hbm, out_hbm, idx_v, rows_v, out_v,
                      sem0, sem1):
    wid = lax.axis_index("s") * 2 + lax.axis_index("c")
    pltpu.sync_copy(idx_hbm.at[wid], idx_v)
    sems = (sem0, sem1)

    def fire(c):
        slot = c % 2
        cps = []
        pos = c * _GROWS
        dst = slot * _GROWS
        while pos < (c + 1) * _GROWS:
            row, off = pos // 128, pos % 128
            n = min(128 - off, (c + 1) * _GROWS - pos)
            src = idx_v.at[row] if (off == 0 and n == 128) \
                else idx_v.at[row].at[pl.ds(off, n)]
            cps.append(
                pltpu.async_copy(
                    tbl_hbm.at[src],
                    rows_v.at[pl.ds(dst, n)],
                    sems[slot],
                )
            )
            pos += n
            dst += n
        return cps

    pending = fire(0)
    for c in range(_NCHUNK):
        nxt = fire(c + 1) if c + 1 < _NCHUNK else []
        for cp in pending:
            cp.wait()
        slot_base = (c % 2) * _GROWS

        def body(r, _):
            for d in range(_DIM // 16):
                acc = rows_v[slot_base + r * _CTX, pl.ds(d * 16, 16)]
                for j in range(1, _CTX):
                    acc = acc + rows_v[slot_base + r * _CTX + j,
                                       pl.ds(d * 16, 16)]
                out_v[c * _CHUNK + r, pl.ds(d * 16, 16)] = acc * (1.0 / _CTX)
            return 0

        lax.fori_loop(0, _CHUNK, body, 0)
        pending = nxt
    pltpu.sync_copy(out_v, out_hbm.at[pl.ds(wid * _BPW, _BPW)])


def _gather_mean(tbl3, x3):
    fn = functools.partial(
        pl.kernel,
        mesh=plsc.VectorSubcoreMesh(core_axis_name="c", subcore_axis_name="s"),
        out_type=jax.ShapeDtypeStruct((_BATCH, _DIM), jnp.float32),
        compiler_params=pltpu.CompilerParams(use_tc_tiling_on_sc=False),
        scratch_types=[
            pltpu.VMEM((_IDXALL, 128), jnp.int32),
            pltpu.VMEM((2 * _GROWS, _PDIM), jnp.float32),
            pltpu.VMEM((_BPW, _DIM), jnp.float32),
            pltpu.SemaphoreType.DMA,
            pltpu.SemaphoreType.DMA,
        ],
    )(_gather_mean_body)
    return fn(tbl3, x3)


_VT = 1024
_NVT = -(-_VOCAB // _VT)


def _linear_body(h_ref, wt_ref, b_ref, out_ref):
    acc = lax.dot_general(
        wt_ref[...], h_ref[...],
        (((0,), (1,)), ((), ())),
        preferred_element_type=jnp.float32,
    )
    bcol = jnp.transpose(b_ref[...])
    out_ref[...] = acc + bcol


def _linear(h, WT, brow):
    return pl.pallas_call(
        _linear_body,
        grid=(_NVT,),
        in_specs=[
            pl.BlockSpec((_BATCH, _DIM), lambda i: (0, 0)),
            pl.BlockSpec((_DIM, _VT), lambda i: (0, i)),
            pl.BlockSpec((1, _VT), lambda i: (0, i)),
        ],
        out_specs=pl.BlockSpec((_VT, _BATCH), lambda i: (i, 0)),
        out_shape=jax.ShapeDtypeStruct((_VOCAB, _BATCH), jnp.float32),
    )(h, WT, brow)


def kernel(x, table, W, b):
    x3 = x.astype(jnp.int32).reshape(_NW, _IDXALL, 128)
    tbl3 = _prep(table.T)
    h = _gather_mean(tbl3, x3)
    outT = _linear(h, W.T, b.reshape(1, _VOCAB))
    return outT.T
